# baseline (speedup 1.0000x reference)
import functools

import jax
import jax.numpy as jnp
from jax import lax
from jax.experimental import pallas as pl
from jax.experimental.pallas import tpu as pltpu
from jax.experimental.pallas import tpu_sc as plsc

N1 = 10000
N2 = 10000
NN = 32
NI = 128
ND = 128
N1R = 10240
L = 16

_PREP_BM = 2000


def _prep_g_body(f2_ref, w1a_ref, g_ref):
    g_ref[...] = jnp.dot(f2_ref[...], w1a_ref[...],
                         preferred_element_type=jnp.float32)


def _prep_g(features2, w1a):
    return pl.pallas_call(
        _prep_g_body,
        grid=(N1 // _PREP_BM,),
        in_specs=[
            pl.BlockSpec((_PREP_BM, NI), lambda j: (j, 0)),
            pl.BlockSpec((NI, ND), lambda j: (0, 0)),
        ],
        out_specs=pl.BlockSpec((_PREP_BM, ND), lambda j: (j, 0)),
        out_shape=jax.ShapeDtypeStruct((N1R, ND), jnp.float32),
    )(features2, w1a)


def _prep_s_body(f1_ref, w1b_ref, b1_ref, s_ref):
    s_ref[...] = (
        jnp.dot(f1_ref[...], w1b_ref[...], preferred_element_type=jnp.float32)
        + b1_ref[...]
    )


def _prep_s(features1, w1b, b1r):
    return pl.pallas_call(
        _prep_s_body,
        grid=(N1 // _PREP_BM,),
        in_specs=[
            pl.BlockSpec((_PREP_BM, NI), lambda j: (j, 0)),
            pl.BlockSpec((NI, ND), lambda j: (0, 0)),
            pl.BlockSpec((1, ND), lambda j: (0, 0)),
        ],
        out_specs=pl.BlockSpec((_PREP_BM, ND), lambda j: (j, 0)),
        out_shape=jax.ShapeDtypeStruct((N1R, ND), jnp.float32),
    )(features1, w1b, b1r)


_CH = 128
_NBUF = 2


def _sc_info():
    info = plsc.get_sparse_core_info()
    return info, info.num_cores * info.num_subcores


def _sc_dis(idx_flat, x2cs, x1cs):
    info, nw = _sc_info()
    epw = (N1R * NN) // nw
    ipw = N1R // nw
    mesh = plsc.VectorSubcoreMesh(core_axis_name="c", subcore_axis_name="s")

    @functools.partial(
        pl.kernel,
        mesh=mesh,
        out_type=jax.ShapeDtypeStruct((N1R * NN,), jnp.float32),
        scratch_types=[
            pltpu.VMEM((epw,), jnp.int32),
            pltpu.VMEM((epw,), jnp.float32),
            pltpu.VMEM((N2,), jnp.float32),
            pltpu.VMEM((N2,), jnp.float32),
            pltpu.VMEM((N2,), jnp.float32),
            pltpu.VMEM((ipw,), jnp.float32),
            pltpu.VMEM((ipw,), jnp.float32),
            pltpu.VMEM((ipw,), jnp.float32),
            pltpu.SemaphoreType.DMA,
        ],
        compiler_params=pltpu.CompilerParams(needs_layout_passes=False),
    )
    def k(idx_hbm, x2x_hbm, x2y_hbm, x2z_hbm, x1x_hbm, x1y_hbm, x1z_hbm,
          dis_hbm, idx_v, disb_v, x2x, x2y, x2z, x1x, x1y, x1z, psem):
        wid = lax.axis_index("s") * info.num_cores + lax.axis_index("c")
        base = wid * epw
        pltpu.async_copy(idx_hbm.at[pl.ds(base, epw)], idx_v, psem)
        pltpu.async_copy(x2x_hbm, x2x, psem)
        pltpu.async_copy(x2y_hbm, x2y, psem)
        pltpu.async_copy(x2z_hbm, x2z, psem)
        pltpu.async_copy(x1x_hbm.at[pl.ds(wid * ipw, ipw)], x1x, psem)
        pltpu.async_copy(x1y_hbm.at[pl.ds(wid * ipw, ipw)], x1y, psem)
        pltpu.async_copy(x1z_hbm.at[pl.ds(wid * ipw, ipw)], x1z, psem)
        pltpu.make_async_copy(idx_hbm.at[pl.ds(base, epw)], idx_v, psem).wait()
        pltpu.make_async_copy(x2x_hbm, x2x, psem).wait()
        pltpu.make_async_copy(x2y_hbm, x2y, psem).wait()
        pltpu.make_async_copy(x2z_hbm, x2z, psem).wait()
        pltpu.make_async_copy(x1x_hbm.at[pl.ds(0, ipw)], x1x, psem).wait()
        pltpu.make_async_copy(x1y_hbm.at[pl.ds(0, ipw)], x1y, psem).wait()
        pltpu.make_async_copy(x1z_hbm.at[pl.ds(0, ipw)], x1z, psem).wait()

        def body(g, carry):
            off = g * L
            iv = idx_v[pl.ds(off, L)]
            gx = plsc.load_gather(x2x, [iv])
            gy = plsc.load_gather(x2y, [iv])
            gz = plsc.load_gather(x2z, [iv])
            ivec = lax.shift_right_logical(
                off + lax.iota(jnp.int32, L), 5)
            sx = plsc.load_gather(x1x, [ivec])
            sy = plsc.load_gather(x1y, [ivec])
            sz = plsc.load_gather(x1z, [ivec])
            dx = gx - sx
            dy = gy - sy
            dz = gz - sz
            d2 = dx * dx + dy * dy + dz * dz
            w = jnp.exp(-0.5 * d2)
            disb_v[pl.ds(off, L)] = jnp.where(iv == 0, 0.0, w)
            return carry

        lax.fori_loop(0, epw // L, body, 0)
        pltpu.sync_copy(disb_v, dis_hbm.at[pl.ds(base, epw)])

    return k(idx_flat, *x2cs, *x1cs)


def _sc_gather(table, idxT_flat, r0, rows_n):
    info, nw = _sc_info()
    nch = rows_n // _CH
    ngrp = nch // _NBUF
    mesh = plsc.VectorSubcoreMesh(core_axis_name="c", subcore_axis_name="s")

    @functools.partial(
        pl.kernel,
        mesh=mesh,
        out_type=jax.ShapeDtypeStruct((NN, rows_n, ND), jnp.float32),
        scratch_types=[
            pltpu.VMEM_SHARED((N1R, ND), jnp.float32),
            pltpu.VMEM((rows_n,), jnp.int32),
            [pltpu.VMEM((_CH, ND), jnp.float32)] * _NBUF,
            [pltpu.SemaphoreType.DMA] * _NBUF,
            [pltpu.SemaphoreType.DMA] * _NBUF,
            pltpu.SemaphoreType.DMA,
        ],
        compiler_params=pltpu.CompilerParams(needs_layout_passes=False),
    )
    def k(t_hbm, idx_hbm, r_hbm, t_sp, idx_v, rows, gsem, wsem, psem):
        wid = lax.axis_index("s") * info.num_cores + lax.axis_index("c")
        base = wid * N1R + r0
        sid = lax.axis_index("s")
        rps = N1R // info.num_subcores
        pltpu.async_copy(idx_hbm.at[pl.ds(base, rows_n)], idx_v, psem)
        pltpu.sync_copy(t_hbm.at[pl.ds(sid * rps, rps)],
                        t_sp.at[pl.ds(sid * rps, rps)])
        pltpu.make_async_copy(idx_hbm.at[pl.ds(base, rows_n)], idx_v,
                              psem).wait()
        plsc.subcore_barrier()

        def _fire_gather(c, b):
            pltpu.async_copy(
                t_sp.at[idx_v.at[pl.ds(c * _CH, _CH)]], rows[b], gsem[b])

        def _wait_gather(b):
            pltpu.make_async_copy(
                t_sp.at[idx_v.at[pl.ds(0, _CH)]], rows[b], gsem[b]).wait()

        def _fire_write(c, b):
            pltpu.async_copy(
                rows[b], r_hbm.at[wid, pl.ds(c * _CH, _CH)], wsem[b])

        def _wait_write(b):
            pltpu.make_async_copy(
                rows[b], r_hbm.at[wid, pl.ds(0, _CH)], wsem[b]).wait()

        for b in range(_NBUF):
            _fire_gather(b, b)

        def body(g, carry):
            for b in range(_NBUF):
                c = g * _NBUF + b
                _wait_gather(b)
                _fire_write(c, b)
                _wait_write(b)
                @pl.when(g < ngrp - 1)
                def _():
                    _fire_gather(c + _NBUF, b)
            return carry

        lax.fori_loop(0, ngrp, body, 0)

    return k(table, idxT_flat)


def _elu(x):
    return jnp.maximum(x, jnp.exp(jnp.minimum(x, 0.0)) - 1.0)


def _main_call(bm, out_rows):
    def body(r_ref, dis_ref, s_ref, w2_ref, b2_ref, o_ref):
        s = s_ref[...]
        w2 = w2_ref[...]
        b2 = b2_ref[...]
        acc = jnp.zeros((bm, ND), jnp.float32)
        for n in range(NN):
            h1 = _elu(r_ref[n] + s)
            h2 = _elu(
                jnp.dot(h1, w2, preferred_element_type=jnp.float32) + b2)
            acc = acc + dis_ref[:, n:n + 1] * h2
        o_ref[...] = acc

    def call(r3, dis2, s, w2, b2r):
        return pl.pallas_call(
            body,
            grid=(out_rows // bm,),
            in_specs=[
                pl.BlockSpec((NN, bm, ND), lambda j: (0, j, 0)),
                pl.BlockSpec((bm, NN), lambda j: (j, 0)),
                pl.BlockSpec((bm, ND), lambda j: (j, 0)),
                pl.BlockSpec((ND, ND), lambda j: (0, 0)),
                pl.BlockSpec((1, ND), lambda j: (0, 0)),
            ],
            out_specs=pl.BlockSpec((bm, ND), lambda j: (j, 0)),
            out_shape=jax.ShapeDtypeStruct((out_rows, ND), jnp.float32),
        )(r3, dis2, s, w2, b2r)

    return call


def kernel(features1, features2, x1, x2, nuv1, nuv2, topk, W1, b1, W2, b2):
    w1a = W1[:NI]
    w1b = W1[NI:]
    b1r = b1.reshape(1, ND)
    b2r = b2.reshape(1, ND)

    topk_p = jnp.pad(topk, ((0, N1R - N1), (0, 0)))
    idx_flat = topk_p.reshape(N1R * NN)
    idxT_flat = topk_p.T.reshape(NN * N1R)
    x2cs = [x2[:, c] for c in range(3)]
    x1p = jnp.pad(x1, ((0, N1R - N1), (0, 0)))
    x1cs = [x1p[:, c] for c in range(3)]

    g_table = _prep_g(features2, w1a)

    parts = [(0, 2560, 2560, 320), (2560, 3840, 3840, 320),
             (6400, 3840, 3600, 400)]
    r3s = [_sc_gather(g_table, idxT_flat, p[0], p[1]) for p in parts[:1]]
    dis = _sc_dis(idx_flat, x2cs, x1cs)
    dis2 = dis.reshape(N1R, NN)
    s = _prep_s(features1, w1b, b1r)
    r3s += [_sc_gather(g_table, idxT_flat, p[0], p[1]) for p in parts[1:]]

    outs = []
    for (r0, g_rows, m_rows, bm), r3 in zip(parts, r3s):
        o = _main_call(bm, m_rows)(
            r3,
            lax.slice(dis2, (r0, 0), (r0 + m_rows, NN)),
            lax.slice(s, (r0, 0), (r0 + m_rows, ND)),
            W2, b2r)
        outs.append(o)
    return jnp.concatenate(outs, axis=0)


# speedup vs baseline: 1.0189x; 1.0189x over previous
import functools

import jax
import jax.numpy as jnp
from jax import lax
from jax.experimental import pallas as pl
from jax.experimental.pallas import tpu as pltpu
from jax.experimental.pallas import tpu_sc as plsc

N1 = 10000
N2 = 10000
NN = 32
NI = 128
ND = 128
N1R = 10240
L = 16

_PREP_BM = 2000


def _prep_g_body(f2_ref, w1a_ref, g_ref):
    g_ref[...] = jnp.dot(f2_ref[...], w1a_ref[...],
                         preferred_element_type=jnp.float32)


def _prep_g(features2, w1a):
    return pl.pallas_call(
        _prep_g_body,
        grid=(N1 // _PREP_BM,),
        in_specs=[
            pl.BlockSpec((_PREP_BM, NI), lambda j: (j, 0)),
            pl.BlockSpec((NI, ND), lambda j: (0, 0)),
        ],
        out_specs=pl.BlockSpec((_PREP_BM, ND), lambda j: (j, 0)),
        out_shape=jax.ShapeDtypeStruct((N1R, ND), jnp.float32),
    )(features2, w1a)


def _prep_s_body(f1_ref, w1b_ref, b1_ref, s_ref):
    s_ref[...] = (
        jnp.dot(f1_ref[...], w1b_ref[...], preferred_element_type=jnp.float32)
        + b1_ref[...]
    )


def _prep_s(features1, w1b, b1r):
    return pl.pallas_call(
        _prep_s_body,
        grid=(N1 // _PREP_BM,),
        in_specs=[
            pl.BlockSpec((_PREP_BM, NI), lambda j: (j, 0)),
            pl.BlockSpec((NI, ND), lambda j: (0, 0)),
            pl.BlockSpec((1, ND), lambda j: (0, 0)),
        ],
        out_specs=pl.BlockSpec((_PREP_BM, ND), lambda j: (j, 0)),
        out_shape=jax.ShapeDtypeStruct((N1R, ND), jnp.float32),
    )(features1, w1b, b1r)


_CH = 128
_NBUF = 2


def _sc_info():
    info = plsc.get_sparse_core_info()
    return info, info.num_cores * info.num_subcores


def _sc_dis(idx_flat, x2cs, x1cs):
    info, nw = _sc_info()
    epw = (N1R * NN) // nw
    ipw = N1R // nw
    mesh = plsc.VectorSubcoreMesh(core_axis_name="c", subcore_axis_name="s")

    @functools.partial(
        pl.kernel,
        mesh=mesh,
        out_type=jax.ShapeDtypeStruct((N1R * NN,), jnp.float32),
        scratch_types=[
            pltpu.VMEM((epw,), jnp.int32),
            pltpu.VMEM((epw,), jnp.float32),
            pltpu.VMEM((N2,), jnp.float32),
            pltpu.VMEM((N2,), jnp.float32),
            pltpu.VMEM((N2,), jnp.float32),
            pltpu.VMEM((ipw,), jnp.float32),
            pltpu.VMEM((ipw,), jnp.float32),
            pltpu.VMEM((ipw,), jnp.float32),
            pltpu.SemaphoreType.DMA,
        ],
        compiler_params=pltpu.CompilerParams(needs_layout_passes=False),
    )
    def k(idx_hbm, x2x_hbm, x2y_hbm, x2z_hbm, x1x_hbm, x1y_hbm, x1z_hbm,
          dis_hbm, idx_v, disb_v, x2x, x2y, x2z, x1x, x1y, x1z, psem):
        wid = lax.axis_index("s") * info.num_cores + lax.axis_index("c")
        base = wid * epw
        pltpu.async_copy(idx_hbm.at[pl.ds(base, epw)], idx_v, psem)
        pltpu.async_copy(x2x_hbm, x2x, psem)
        pltpu.async_copy(x2y_hbm, x2y, psem)
        pltpu.async_copy(x2z_hbm, x2z, psem)
        pltpu.async_copy(x1x_hbm.at[pl.ds(wid * ipw, ipw)], x1x, psem)
        pltpu.async_copy(x1y_hbm.at[pl.ds(wid * ipw, ipw)], x1y, psem)
        pltpu.async_copy(x1z_hbm.at[pl.ds(wid * ipw, ipw)], x1z, psem)
        pltpu.make_async_copy(idx_hbm.at[pl.ds(base, epw)], idx_v, psem).wait()
        pltpu.make_async_copy(x2x_hbm, x2x, psem).wait()
        pltpu.make_async_copy(x2y_hbm, x2y, psem).wait()
        pltpu.make_async_copy(x2z_hbm, x2z, psem).wait()
        pltpu.make_async_copy(x1x_hbm.at[pl.ds(0, ipw)], x1x, psem).wait()
        pltpu.make_async_copy(x1y_hbm.at[pl.ds(0, ipw)], x1y, psem).wait()
        pltpu.make_async_copy(x1z_hbm.at[pl.ds(0, ipw)], x1z, psem).wait()

        def body(g, carry):
            off = g * L
            iv = idx_v[pl.ds(off, L)]
            gx = plsc.load_gather(x2x, [iv])
            gy = plsc.load_gather(x2y, [iv])
            gz = plsc.load_gather(x2z, [iv])
            ivec = lax.shift_right_logical(
                off + lax.iota(jnp.int32, L), 5)
            sx = plsc.load_gather(x1x, [ivec])
            sy = plsc.load_gather(x1y, [ivec])
            sz = plsc.load_gather(x1z, [ivec])
            dx = gx - sx
            dy = gy - sy
            dz = gz - sz
            d2 = dx * dx + dy * dy + dz * dz
            w = jnp.exp(-0.5 * d2)
            disb_v[pl.ds(off, L)] = jnp.where(iv == 0, 0.0, w)
            return carry

        lax.fori_loop(0, epw // L, body, 0)
        pltpu.sync_copy(disb_v, dis_hbm.at[pl.ds(base, epw)])

    return k(idx_flat, *x2cs, *x1cs)


def _sc_gather(table, idxT_flat, r0, rows_n):
    info, nw = _sc_info()
    nch = rows_n // _CH
    ngrp = nch // _NBUF
    mesh = plsc.VectorSubcoreMesh(core_axis_name="c", subcore_axis_name="s")

    @functools.partial(
        pl.kernel,
        mesh=mesh,
        out_type=jax.ShapeDtypeStruct((NN, rows_n, ND), jnp.float32),
        scratch_types=[
            pltpu.VMEM_SHARED((N1R, ND), jnp.float32),
            pltpu.VMEM((rows_n,), jnp.int32),
            [pltpu.VMEM((_CH, ND), jnp.float32)] * _NBUF,
            [pltpu.SemaphoreType.DMA] * _NBUF,
            [pltpu.SemaphoreType.DMA] * _NBUF,
            pltpu.SemaphoreType.DMA,
        ],
        compiler_params=pltpu.CompilerParams(needs_layout_passes=False),
    )
    def k(t_hbm, idx_hbm, r_hbm, t_sp, idx_v, rows, gsem, wsem, psem):
        wid = lax.axis_index("s") * info.num_cores + lax.axis_index("c")
        base = wid * N1R + r0
        sid = lax.axis_index("s")
        rps = N1R // info.num_subcores
        pltpu.async_copy(idx_hbm.at[pl.ds(base, rows_n)], idx_v, psem)
        pltpu.sync_copy(t_hbm.at[pl.ds(sid * rps, rps)],
                        t_sp.at[pl.ds(sid * rps, rps)])
        pltpu.make_async_copy(idx_hbm.at[pl.ds(base, rows_n)], idx_v,
                              psem).wait()
        plsc.subcore_barrier()

        def _fire_gather(c, b):
            pltpu.async_copy(
                t_sp.at[idx_v.at[pl.ds(c * _CH, _CH)]], rows[b], gsem[b])

        def _wait_gather(b):
            pltpu.make_async_copy(
                t_sp.at[idx_v.at[pl.ds(0, _CH)]], rows[b], gsem[b]).wait()

        def _fire_write(c, b):
            pltpu.async_copy(
                rows[b], r_hbm.at[wid, pl.ds(c * _CH, _CH)], wsem[b])

        def _wait_write(b):
            pltpu.make_async_copy(
                rows[b], r_hbm.at[wid, pl.ds(0, _CH)], wsem[b]).wait()

        for b in range(_NBUF):
            _fire_gather(b, b)

        def body(g, carry):
            for b in range(_NBUF):
                c = g * _NBUF + b
                _wait_gather(b)
                _fire_write(c, b)
                _wait_write(b)
                @pl.when(g < ngrp - 1)
                def _():
                    _fire_gather(c + _NBUF, b)
            return carry

        lax.fori_loop(0, ngrp, body, 0)

    return k(table, idxT_flat)


def _elu(x):
    return jnp.maximum(x, jnp.exp(jnp.minimum(x, 0.0)) - 1.0)


def _main_call(bm, out_rows):
    def body(r_ref, dis_ref, s_ref, w2_ref, b2_ref, o_ref):
        s = s_ref[...]
        w2 = w2_ref[...]
        b2 = b2_ref[...]
        acc = jnp.zeros((bm, ND), jnp.float32)
        for n in range(NN):
            h1 = _elu(r_ref[n] + s)
            h2 = _elu(
                jnp.dot(h1, w2, preferred_element_type=jnp.float32) + b2)
            acc = acc + dis_ref[:, n:n + 1] * h2
        o_ref[...] = acc

    def call(r3, dis2, s, w2, b2r):
        return pl.pallas_call(
            body,
            grid=(out_rows // bm,),
            in_specs=[
                pl.BlockSpec((NN, bm, ND), lambda j: (0, j, 0)),
                pl.BlockSpec((bm, NN), lambda j: (j, 0)),
                pl.BlockSpec((bm, ND), lambda j: (j, 0)),
                pl.BlockSpec((ND, ND), lambda j: (0, 0)),
                pl.BlockSpec((1, ND), lambda j: (0, 0)),
            ],
            out_specs=pl.BlockSpec((bm, ND), lambda j: (j, 0)),
            out_shape=jax.ShapeDtypeStruct((out_rows, ND), jnp.float32),
        )(r3, dis2, s, w2, b2r)

    return call


def kernel(features1, features2, x1, x2, nuv1, nuv2, topk, W1, b1, W2, b2):
    w1a = W1[:NI]
    w1b = W1[NI:]
    b1r = b1.reshape(1, ND)
    b2r = b2.reshape(1, ND)

    topk_p = jnp.pad(topk, ((0, N1R - N1), (0, 0)))
    idx_flat = topk_p.reshape(N1R * NN)
    idxT_flat = topk_p.T.reshape(NN * N1R)
    x2cs = [x2[:, c] for c in range(3)]
    x1p = jnp.pad(x1, ((0, N1R - N1), (0, 0)))
    x1cs = [x1p[:, c] for c in range(3)]

    g_table = _prep_g(features2, w1a)

    parts = [(0, 4096, 4096, 512), (4096, 6144, 5904, 328)]
    r3s = [_sc_gather(g_table, idxT_flat, p[0], p[1]) for p in parts[:1]]
    dis = _sc_dis(idx_flat, x2cs, x1cs)
    dis2 = dis.reshape(N1R, NN)
    s = _prep_s(features1, w1b, b1r)
    r3s += [_sc_gather(g_table, idxT_flat, p[0], p[1]) for p in parts[1:]]

    outs = []
    for (r0, g_rows, m_rows, bm), r3 in zip(parts, r3s):
        o = _main_call(bm, m_rows)(
            r3,
            lax.slice(dis2, (r0, 0), (r0 + m_rows, NN)),
            lax.slice(s, (r0, 0), (r0 + m_rows, ND)),
            W2, b2r)
        outs.append(o)
    return jnp.concatenate(outs, axis=0)
